# Initial kernel scaffold; baseline (speedup 1.0000x reference)
import jax
import jax.numpy as jnp
from jax.experimental import pallas as pl


def kernel(x, lookup_table):
    raise NotImplementedError("write your pallas kernel here")


# speedup vs baseline: 4.7262x; 4.7262x over previous
import functools
import math

import jax
import jax.numpy as jnp
from jax import lax
from jax.experimental import pallas as pl
from jax.experimental.pallas import tpu as pltpu
from jax.experimental.pallas import tpu_sc as plsc

_D = 128
_LANES = 16
_NC, _NS = 2, 16
_NW = _NC * _NS
_SCALE = math.sqrt(_D)


def _make_kernel(batch: int):
    b_per_w = batch // _NW
    chunk = 128
    n_chunks = b_per_w // chunk

    mesh = plsc.VectorSubcoreMesh(
        core_axis_name="c", subcore_axis_name="s",
        num_cores=_NC, num_subcores=_NS,
    )

    @functools.partial(
        pl.kernel,
        out_type=jax.ShapeDtypeStruct((batch, _D), jnp.float32),
        mesh=mesh,
        scratch_types=[
            pltpu.VMEM((b_per_w,), jnp.int32),
            pltpu.VMEM((chunk, _D), jnp.float32),
            pltpu.SemaphoreType.DMA,
        ],
    )
    def emb(idx_hbm, table_hbm, out_hbm, idx_v, rows_v, sem):
        wid = lax.axis_index("s") * _NC + lax.axis_index("c")
        base = wid * b_per_w
        pltpu.sync_copy(idx_hbm.at[pl.ds(base, b_per_w)], idx_v)

        def do_chunk(c, carry):
            pltpu.async_copy(
                table_hbm.at[idx_v.at[pl.ds(c * chunk, chunk)]], rows_v, sem
            ).wait()

            def do_row(r, carry2):
                for j in range(_D // _LANES):
                    sl = pl.ds(j * _LANES, _LANES)
                    rows_v[r, sl] = rows_v[r, sl] * _SCALE
                return carry2

            lax.fori_loop(0, chunk, do_row, 0)
            pltpu.sync_copy(rows_v, out_hbm.at[pl.ds(base + c * chunk, chunk)])
            return carry

        lax.fori_loop(0, n_chunks, do_chunk, 0)

    return emb


def kernel(x, lookup_table):
    batch, seq = x.shape
    idx = x.reshape(batch * seq).astype(jnp.int32)
    out = _make_kernel(batch * seq)(idx, lookup_table)
    return out.reshape(batch, seq, _D)


# speedup vs baseline: 7.8305x; 1.6568x over previous
import functools
import math

import jax
import jax.numpy as jnp
from jax import lax
from jax.experimental import pallas as pl
from jax.experimental.pallas import tpu as pltpu
from jax.experimental.pallas import tpu_sc as plsc

_D = 128
_LANES = 16
_NC, _NS = 2, 16
_NW = _NC * _NS
_SCALE = math.sqrt(_D)
_CH = 128


def _make_kernel(batch: int):
    b_per_w = batch // _NW
    n_chunks = b_per_w // _CH
    n_pairs = n_chunks // 2
    assert n_chunks % 2 == 0 and n_pairs >= 2

    mesh = plsc.VectorSubcoreMesh(
        core_axis_name="c", subcore_axis_name="s",
        num_cores=_NC, num_subcores=_NS,
    )

    @functools.partial(
        pl.kernel,
        out_type=jax.ShapeDtypeStruct((batch, _D), jnp.float32),
        mesh=mesh,
        scratch_types=[
            pltpu.VMEM((b_per_w,), jnp.int32),
            pltpu.VMEM((_CH, _D), jnp.float32),
            pltpu.VMEM((_CH, _D), jnp.float32),
            pltpu.VMEM((_CH, _D), jnp.float32),
            pltpu.VMEM((_CH, _D), jnp.float32),
            pltpu.SemaphoreType.DMA,
            pltpu.SemaphoreType.DMA,
            pltpu.SemaphoreType.DMA,
            pltpu.SemaphoreType.DMA,
        ],
    )
    def emb(idx_hbm, table_hbm, out_hbm, idx_v,
            g0, g1, w0, w1, gs0, gs1, ws0, ws1):
        wid = lax.axis_index("s") * _NC + lax.axis_index("c")
        base = wid * b_per_w
        pltpu.sync_copy(idx_hbm.at[pl.ds(base, b_per_w)], idx_v)

        gbuf = (g0, g1)
        wbuf = (w0, w1)
        gsem = (gs0, gs1)
        wsem = (ws0, ws1)

        def start_gather(c, b):
            pltpu.async_copy(
                table_hbm.at[idx_v.at[pl.ds(c * _CH, _CH)]], gbuf[b], gsem[b])

        def wait_gather(b):
            pltpu.make_async_copy(
                table_hbm.at[idx_v.at[pl.ds(0, _CH)]], gbuf[b], gsem[b]).wait()

        def start_write(c, b):
            pltpu.async_copy(
                wbuf[b], out_hbm.at[pl.ds(base + c * _CH, _CH)], wsem[b])

        def wait_write(b):
            pltpu.make_async_copy(
                wbuf[b], out_hbm.at[pl.ds(0, _CH)], wsem[b]).wait()

        def scale(b):
            g, w = gbuf[b], wbuf[b]

            def do_row(r, carry):
                for j in range(_D // _LANES):
                    sl = pl.ds(j * _LANES, _LANES)
                    w[r, sl] = g[r, sl] * _SCALE
                return carry

            lax.fori_loop(0, _CH, do_row, 0)

        start_gather(0, 0)
        start_gather(1, 1)
        for b in range(2):
            wait_gather(b)
            scale(b)
            start_write(b, b)
            start_gather(b + 2, b)

        def pair_body(p, carry):
            c = 2 * p
            for b in range(2):
                wait_gather(b)
                wait_write(b)
                scale(b)
                start_write(c + b, b)
                start_gather(c + b + 2, b)
            return carry

        lax.fori_loop(1, n_pairs - 1, pair_body, 0)

        c = n_chunks - 2
        for b in range(2):
            wait_gather(b)
            wait_write(b)
            scale(b)
            start_write(c + b, b)
        for b in range(2):
            wait_write(b)

    return emb


def kernel(x, lookup_table):
    batch, seq = x.shape
    idx = x.reshape(batch * seq).astype(jnp.int32)
    out = _make_kernel(batch * seq)(idx, lookup_table)
    return out.reshape(batch, seq, _D)
